# Initial kernel scaffold; baseline (speedup 1.0000x reference)
import jax
import jax.numpy as jnp
from jax.experimental import pallas as pl


def kernel(x, edge_index, pairs, Wl1, Wr1, att1, bc1, Wl2, Wr2, att2, bc2, W1, b1, W2, b2):
    raise NotImplementedError("write your pallas kernel here")


# speedup vs baseline: 4.2511x; 4.2511x over previous
import functools

import jax
import jax.numpy as jnp
from jax import lax
from jax.experimental import pallas as pl
from jax.experimental.pallas import tpu as pltpu
from jax.experimental.pallas import tpu_sc as plsc

N = 10000
E = 160000
D = 128
C = 250
H1 = 256
P = 65536

NPAD = 10240
CP = 256
EP = 172032
DEN_CH = C - 128
DUMMY = N

NC, NS = 2, 16
NW = NC * NS

TE1 = EP // NW
G1 = 128
CH1 = TE1 // G1

TE2 = EP // NS
G2 = 128
CH2 = TE2 // G2

TP = P // NW
G3 = 128
CH3 = TP // G3

ROWS_PER_TILE = NPAD // NS


def _leaky(v):
    return jnp.maximum(v, 0.2 * v)


def _t_proj_body(x_ref, wl_ref, wr_ref, ll_ref, lh_ref, rl_ref, rh_ref):
    xb = x_ref[...]
    xl = jnp.dot(xb, wl_ref[...], preferred_element_type=jnp.float32)
    xr = jnp.dot(xb, wr_ref[...], preferred_element_type=jnp.float32)
    ll_ref[...] = xl[:, :128]
    lh_ref[...] = xl[:, 128:]
    rl_ref[...] = xr[:, :128]
    rh_ref[...] = xr[:, 128:]


def _t_proj(x_pad, wl, wr):
    return pl.pallas_call(
        _t_proj_body,
        grid=(NPAD // 256,),
        in_specs=[
            pl.BlockSpec((256, D), lambda i: (i, 0)),
            pl.BlockSpec((D, CP), lambda i: (0, 0)),
            pl.BlockSpec((D, CP), lambda i: (0, 0)),
        ],
        out_specs=[pl.BlockSpec((256, 128), lambda i: (i, 0))] * 4,
        out_shape=[jax.ShapeDtypeStruct((NPAD, 128), jnp.float32)] * 4,
    )(x_pad, wl, wr)


def _group_sum_mat():
    r = lax.broadcasted_iota(jnp.int32, (128, 8), 0)
    cc = lax.broadcasted_iota(jnp.int32, (128, 8), 1)
    return jnp.where(r // 16 == cc, 1.0, 0.0).astype(jnp.float32)


def _t_exp_body(l_ref, w_ref):
    l = jnp.dot(l_ref[...], _group_sum_mat(),
                preferred_element_type=jnp.float32)
    m = jnp.max(l)
    w_ref[...] = jnp.exp(l - m)


def _t_exp(logits16):
    l2 = logits16.reshape(EP // 8, 128)
    w = pl.pallas_call(
        _t_exp_body,
        out_shape=jax.ShapeDtypeStruct((EP // 8, 8), jnp.float32),
    )(l2)
    return w.reshape(EP)


def _t_norm_proj_body(relu_out, ol_ref, oh_ref, b_ref, wl_ref, wr_ref,
                      ll_ref, lh_ref, rl_ref, rh_ref):
    lo = ol_ref[...]
    hi = oh_ref[...]
    den = hi[:, DEN_CH:DEN_CH + 1] + 1e-16
    o = jnp.concatenate([lo, hi], axis=1) / den + b_ref[...]
    if relu_out:
        o = jnp.maximum(o, 0.0)
    mask = (lax.broadcasted_iota(jnp.int32, (1, CP), 1) < C)
    h = jnp.where(mask, o, 0.0)
    xl = jnp.dot(h, wl_ref[...], preferred_element_type=jnp.float32)
    xr = jnp.dot(h, wr_ref[...], preferred_element_type=jnp.float32)
    ll_ref[...] = xl[:, :128]
    lh_ref[...] = xl[:, 128:]
    rl_ref[...] = xr[:, :128]
    rh_ref[...] = xr[:, 128:]


def _t_norm_proj(relu_out, o_acc, bias_row, wl, wr):
    return pl.pallas_call(
        functools.partial(_t_norm_proj_body, relu_out),
        grid=(NPAD // 256,),
        in_specs=[
            pl.BlockSpec((256, 128), lambda i: (i, 0)),
            pl.BlockSpec((256, 128), lambda i: (i, 0)),
            pl.BlockSpec((1, CP), lambda i: (0, 0)),
            pl.BlockSpec((CP, CP), lambda i: (0, 0)),
            pl.BlockSpec((CP, CP), lambda i: (0, 0)),
        ],
        out_specs=[pl.BlockSpec((256, 128), lambda i: (i, 0))] * 4,
        out_shape=[jax.ShapeDtypeStruct((NPAD, 128), jnp.float32)] * 4,
    )(o_acc[0], o_acc[1], bias_row, wl, wr)


def _t_pair_proj_body(ol_ref, oh_ref, b_ref, w1a_ref, w1b_ref, b1_ref,
                      z0_ref, z1_ref):
    lo = ol_ref[...]
    hi = oh_ref[...]
    den = hi[:, DEN_CH:DEN_CH + 1] + 1e-16
    o = jnp.concatenate([lo, hi], axis=1) / den + b_ref[...]
    mask = (lax.broadcasted_iota(jnp.int32, (1, CP), 1) < C)
    emb = jnp.where(mask, o, 0.0)
    z0_ref[...] = jnp.dot(emb, w1a_ref[...],
                          preferred_element_type=jnp.float32) + b1_ref[...]
    z1_ref[...] = jnp.dot(emb, w1b_ref[...],
                          preferred_element_type=jnp.float32)


def _t_pair_proj(o_acc, bias_row, w1a, w1b, b1_row):
    return pl.pallas_call(
        _t_pair_proj_body,
        grid=(NPAD // 256,),
        in_specs=[
            pl.BlockSpec((256, 128), lambda i: (i, 0)),
            pl.BlockSpec((256, 128), lambda i: (i, 0)),
            pl.BlockSpec((1, CP), lambda i: (0, 0)),
            pl.BlockSpec((CP, H1), lambda i: (0, 0)),
            pl.BlockSpec((CP, H1), lambda i: (0, 0)),
            pl.BlockSpec((1, H1), lambda i: (0, 0)),
        ],
        out_specs=[pl.BlockSpec((256, H1), lambda i: (i, 0))] * 2,
        out_shape=[jax.ShapeDtypeStruct((NPAD, H1), jnp.float32)] * 2,
    )(o_acc[0], o_acc[1], bias_row, w1a, w1b, b1_row)


def _t_final_body(s_ref, b2_ref, o_ref):
    ss = jnp.dot(s_ref[...], _group_sum_mat(),
                 preferred_element_type=jnp.float32)
    o_ref[...] = jnp.maximum(ss + b2_ref[...], 0.0)


def _t_final(s16, b2):
    s2 = s16.reshape(P // 8, 128)
    b2r = jnp.broadcast_to(b2.reshape(1, 1), (1, 8))
    out = pl.pallas_call(
        _t_final_body,
        out_shape=jax.ShapeDtypeStruct((P // 8, 8), jnp.float32),
    )(s2, b2r)
    return out.reshape(P, 1)


_MESH = plsc.VectorSubcoreMesh(core_axis_name="c", subcore_axis_name="s")


@functools.partial(
    pl.kernel,
    mesh=_MESH,
    out_type=jax.ShapeDtypeStruct((EP, 16), jnp.float32),
    scratch_types=[
        pltpu.VMEM((TE1,), jnp.int32),
        pltpu.VMEM((TE1,), jnp.int32),
        pltpu.VMEM((G1, 16), jnp.float32),
        pltpu.VMEM((G1, 128), jnp.float32),
        pltpu.VMEM((G1, 128), jnp.float32),
        pltpu.VMEM((G1, 128), jnp.float32),
        pltpu.VMEM((G1, 128), jnp.float32),
        pltpu.VMEM((CP,), jnp.float32),
        pltpu.SemaphoreType.DMA,
        pltpu.SemaphoreType.DMA,
        pltpu.SemaphoreType.DMA,
        pltpu.SemaphoreType.DMA,
    ],
)
def _sc_logits(xl_lo, xl_hi, xr_lo, xr_hi, src_hbm, dst_hbm, att_hbm,
               out_hbm, src_v, dst_v, lg_c, bll, blh, brl, brh, att_v,
               sem0, sem1, sem2, sem3):
    c = lax.axis_index("c")
    s = lax.axis_index("s")
    wid = s * NC + c
    base = wid * TE1
    pltpu.sync_copy(src_hbm.at[pl.ds(base, TE1)], src_v)
    pltpu.sync_copy(dst_hbm.at[pl.ds(base, TE1)], dst_v)
    pltpu.sync_copy(att_hbm, att_v)

    def chunk(ci, carry):
        off = ci * G1
        cp0 = pltpu.async_copy(xl_lo.at[src_v.at[pl.ds(off, G1)]], bll, sem0)
        cp1 = pltpu.async_copy(xl_hi.at[src_v.at[pl.ds(off, G1)]], blh, sem1)
        cp2 = pltpu.async_copy(xr_lo.at[dst_v.at[pl.ds(off, G1)]], brl, sem2)
        cp3 = pltpu.async_copy(xr_hi.at[dst_v.at[pl.ds(off, G1)]], brh, sem3)
        cp0.wait()
        cp1.wait()
        cp2.wait()
        cp3.wait()

        def edge(e, carry2):
            acc = jnp.zeros((16,), jnp.float32)
            for k in range(8):
                sl = pl.ds(16 * k, 16)
                sh = pl.ds(128 + 16 * k, 16)
                v = bll[e, sl] + brl[e, sl]
                acc = acc + att_v[sl] * _leaky(v)
                v2 = blh[e, sl] + brh[e, sl]
                acc = acc + att_v[sh] * _leaky(v2)
            lg_c[e, :] = acc
            return carry2

        lax.fori_loop(0, G1, edge, 0)
        pltpu.sync_copy(lg_c, out_hbm.at[pl.ds(base + off, G1)])
        return carry

    lax.fori_loop(0, CH1, chunk, 0)


@functools.partial(
    pl.kernel,
    mesh=_MESH,
    out_type=jax.ShapeDtypeStruct((NC, NPAD, 128), jnp.float32),
    scratch_types=[
        pltpu.VMEM((TE2,), jnp.int32),
        pltpu.VMEM((CH2, G2), jnp.int32),
        pltpu.VMEM((TE2,), jnp.float32),
        pltpu.VMEM((G2, 128), jnp.float32),
        pltpu.VMEM_SHARED((NPAD, 128), jnp.float32),
        pltpu.SemaphoreType.DMA,
    ],
)
def _sc_scatter(xl_lo, xl_hi, src_hbm, dst3d_hbm, w_hbm, out_hbm,
                src_v, dst_v, w_v, rows, shared, sem):
    c = lax.axis_index("c")
    s = lax.axis_index("s")
    ebase = s * TE2
    pltpu.sync_copy(src_hbm.at[pl.ds(ebase, TE2)], src_v)
    pltpu.sync_copy(dst3d_hbm.at[s], dst_v)
    pltpu.sync_copy(w_hbm.at[pl.ds(ebase, TE2)], w_v)

    def zrow(e, carry):
        for k in range(8):
            rows[e, pl.ds(16 * k, 16)] = jnp.zeros((16,), jnp.float32)
        return carry

    lax.fori_loop(0, G2, zrow, 0)
    nbase = s * ROWS_PER_TILE
    for i in range(ROWS_PER_TILE // G2):
        pltpu.sync_copy(rows, shared.at[pl.ds(nbase + i * G2, G2)])
    plsc.subcore_barrier()

    lane = lax.broadcasted_iota(jnp.int32, (16,), 0)

    def chunk_body(tab, is_hi, ci):
        off = ci * G2
        pltpu.async_copy(tab.at[src_v.at[pl.ds(off, G2)]], rows, sem).wait()

        def grp(g, carry2):
            wrow = w_v[pl.ds(off + g * 16, 16)]
            for j in range(16):
                e = g * 16 + j
                wv = wrow[j]
                for k in range(8):
                    sl = pl.ds(16 * k, 16)
                    scaled = rows[e, sl] * wv
                    if is_hi and k == 7:
                        scaled = jnp.where(lane == (DEN_CH - 112), wv, scaled)
                    rows[e, sl] = scaled
            return carry2

        lax.fori_loop(0, G2 // 16, grp, 0)
        pltpu.sync_copy(rows, shared.at[dst_v.at[ci]], add=True)

    @pl.when(c == 0)
    def _():
        def chunk0(ci, carry):
            chunk_body(xl_lo, False, ci)
            return carry
        lax.fori_loop(0, CH2, chunk0, 0)

    @pl.when(c == 1)
    def _():
        def chunk1(ci, carry):
            chunk_body(xl_hi, True, ci)
            return carry
        lax.fori_loop(0, CH2, chunk1, 0)

    plsc.subcore_barrier()
    pltpu.sync_copy(shared.at[pl.ds(nbase, ROWS_PER_TILE)],
                    out_hbm.at[c].at[pl.ds(nbase, ROWS_PER_TILE)])


@functools.partial(
    pl.kernel,
    mesh=_MESH,
    out_type=jax.ShapeDtypeStruct((P, 16), jnp.float32),
    scratch_types=[
        pltpu.VMEM((TP,), jnp.int32),
        pltpu.VMEM((TP,), jnp.int32),
        pltpu.VMEM((G3, 16), jnp.float32),
        pltpu.VMEM((G3, CP), jnp.float32),
        pltpu.VMEM((G3, CP), jnp.float32),
        pltpu.VMEM((CP,), jnp.float32),
        pltpu.SemaphoreType.DMA,
        pltpu.SemaphoreType.DMA,
    ],
)
def _sc_pairs(z0_hbm, z1_hbm, p0_hbm, p1_hbm, w2_hbm, out_hbm,
              p0_v, p1_v, o_c, r0, r1, w2_v, sem0, sem1):
    c = lax.axis_index("c")
    s = lax.axis_index("s")
    wid = s * NC + c
    base = wid * TP
    pltpu.sync_copy(p0_hbm.at[pl.ds(base, TP)], p0_v)
    pltpu.sync_copy(p1_hbm.at[pl.ds(base, TP)], p1_v)
    pltpu.sync_copy(w2_hbm, w2_v)

    def chunk(ci, carry):
        off = ci * G3
        cp0 = pltpu.async_copy(z0_hbm.at[p0_v.at[pl.ds(off, G3)]], r0, sem0)
        cp1 = pltpu.async_copy(z1_hbm.at[p1_v.at[pl.ds(off, G3)]], r1, sem1)
        cp0.wait()
        cp1.wait()

        def pair(e, carry2):
            acc = jnp.zeros((16,), jnp.float32)
            for k in range(16):
                sl = pl.ds(16 * k, 16)
                v = jnp.maximum(r0[e, sl] + r1[e, sl], 0.0)
                acc = acc + w2_v[sl] * v
            o_c[e, :] = acc
            return carry2

        lax.fori_loop(0, G3, pair, 0)
        pltpu.sync_copy(o_c, out_hbm.at[pl.ds(base + off, G3)])
        return carry

    lax.fori_loop(0, CH3, chunk, 0)


def kernel(x, edge_index, pairs, Wl1, Wr1, att1, bc1, Wl2, Wr2, att2, bc2,
           W1, b1, W2, b2):
    loop = jnp.arange(N, dtype=jnp.int32)
    padi = jnp.full((EP - E - N,), DUMMY, jnp.int32)
    src = jnp.concatenate([edge_index[0], loop, padi])
    dst = jnp.concatenate([edge_index[1], loop, padi])
    dst3d = dst.reshape(NS, CH2, G2)
    x_pad = jnp.pad(x, ((0, NPAD - N), (0, 0)))

    def padw(w):
        return jnp.pad(w, ((0, 0), (0, CP - C)))

    wl1 = padw(Wl1)
    wr1 = padw(Wr1)
    att1p = jnp.pad(att1, (0, CP - C))
    att2p = jnp.pad(att2, (0, CP - C))
    bc1r = jnp.pad(bc1, (0, CP - C)).reshape(1, CP)
    bc2r = jnp.pad(bc2, (0, CP - C)).reshape(1, CP)
    wl2 = jnp.pad(Wl2, ((0, CP - C), (0, CP - C)))
    wr2 = jnp.pad(Wr2, ((0, CP - C), (0, CP - C)))
    w1a = jnp.pad(W1[:C], ((0, CP - C), (0, 0)))
    w1b = jnp.pad(W1[C:], ((0, CP - C), (0, 0)))
    b1r = b1.reshape(1, H1)
    w2v = W2[:, 0]
    p0 = pairs[:, 0]
    p1 = pairs[:, 1]

    ll1, lh1, rl1, rh1 = _t_proj(x_pad, wl1, wr1)
    logits1 = _sc_logits(ll1, lh1, rl1, rh1, src, dst, att1p)
    w1e = _t_exp(logits1)
    acc1 = _sc_scatter(ll1, lh1, src, dst3d, w1e)

    ll2, lh2, rl2, rh2 = _t_norm_proj(True, acc1, bc1r, wl2, wr2)
    logits2 = _sc_logits(ll2, lh2, rl2, rh2, src, dst, att2p)
    w2e = _t_exp(logits2)
    acc2 = _sc_scatter(ll2, lh2, src, dst3d, w2e)

    z0, z1 = _t_pair_proj(acc2, bc2r, w1a, w1b, b1r)
    s16 = _sc_pairs(z0, z1, p0, p1, w2v)
    return _t_final(s16, b2)


# speedup vs baseline: 5.3776x; 1.2650x over previous
import functools

import jax
import jax.numpy as jnp
from jax import lax
from jax.experimental import pallas as pl
from jax.experimental.pallas import tpu as pltpu
from jax.experimental.pallas import tpu_sc as plsc

N = 10000
E = 160000
D = 128
C = 250
H1 = 256
P = 65536

NPAD = 10240
CP = 256
EP = 172032
DEN_CH = C - 128
DUMMY = N

NC, NS = 2, 16
NW = NC * NS

TE1 = EP // NW
G1 = 64
CH1 = TE1 // G1

TE2 = EP // NS
G2 = 128
CH2 = TE2 // G2

TP = P // NW
G3 = 64
CH3 = TP // G3

ROWS_PER_TILE = NPAD // NS


def _leaky(v):
    return jnp.maximum(v, 0.2 * v)


def _round_bf16(v):
    u = lax.bitcast_convert_type(v, jnp.int32)
    u = u + 0x7FFF + ((u >> 16) & 1)
    u = lax.bitwise_and(u, jnp.int32(-65536))
    return lax.bitcast_convert_type(u, jnp.float32)


def _dot3(a, b):
    f32 = jnp.float32
    ah = a.astype(jnp.bfloat16)
    al = (a - ah.astype(f32)).astype(jnp.bfloat16)
    bh = b.astype(jnp.bfloat16)
    bl = (b - bh.astype(f32)).astype(jnp.bfloat16)

    def d(u, v):
        return lax.dot_general(u, v, (((1,), (0,)), ((), ())),
                               preferred_element_type=f32)

    return d(ah, bh) + (d(ah, bl) + d(al, bh))


def _dot_bf16(a, b):
    return lax.dot_general(a.astype(jnp.bfloat16), b.astype(jnp.bfloat16),
                           (((1,), (0,)), ((), ())),
                           preferred_element_type=jnp.float32)


def _precise_recip(den):
    r = 1.0 / den
    return r * (2.0 - den * r)


def _t_proj_body(x_ref, wl_ref, wr_ref, ll_ref, lh_ref, rl_ref, rh_ref):
    xb = x_ref[...]
    xl = _dot_bf16(xb, wl_ref[...])
    xr = _dot_bf16(xb, wr_ref[...])
    ll_ref[...] = xl[:, :128]
    lh_ref[...] = xl[:, 128:]
    rl_ref[...] = xr[:, :128]
    rh_ref[...] = xr[:, 128:]


def _t_proj(x_pad, wl, wr):
    return pl.pallas_call(
        _t_proj_body,
        grid=(NPAD // 256,),
        in_specs=[
            pl.BlockSpec((256, D), lambda i: (i, 0)),
            pl.BlockSpec((D, CP), lambda i: (0, 0)),
            pl.BlockSpec((D, CP), lambda i: (0, 0)),
        ],
        out_specs=[pl.BlockSpec((256, 128), lambda i: (i, 0))] * 4,
        out_shape=[jax.ShapeDtypeStruct((NPAD, 128), jnp.float32)] * 4,
    )(x_pad, wl, wr)


def _group_sum_mat():
    r = lax.broadcasted_iota(jnp.int32, (128, 8), 0)
    cc = lax.broadcasted_iota(jnp.int32, (128, 8), 1)
    return jnp.where(r // 16 == cc, 1.0, 0.0).astype(jnp.float32)


def _t_gsum_body(l_ref, s_ref):
    s_ref[...] = _dot3(l_ref[...], _group_sum_mat())


def _exp_sw(x):
    y = jnp.maximum(x, -86.0) * 1.4426950408889634
    yi = jnp.round(y)
    yf = y - yi
    p = jnp.float32(0.00015403530393381608)
    for cc in (0.0013333558146428443, 0.009618129107628477,
               0.05550410866482158, 0.2402265069591007,
               0.6931471805599453, 1.0):
        p = p * yf + cc
    ex = lax.bitcast_convert_type((yi.astype(jnp.int32) + 127) << 23,
                                  jnp.float32)
    return p * ex


def _t_expmax_body(s_ref, w_ref, m_ref):
    s = s_ref[...]
    m = jnp.max(s)
    w_ref[...] = jnp.exp(s - m)
    m_ref[...] = jnp.full((1, 8), m, jnp.float32)


def _t_exp(logits16):
    l2 = logits16.reshape(EP // 8, 128)
    s = pl.pallas_call(
        _t_gsum_body,
        grid=(8,),
        in_specs=[pl.BlockSpec((EP // 64, 128), lambda i: (i, 0))],
        out_specs=pl.BlockSpec((EP // 64, 8), lambda i: (i, 0)),
        out_shape=jax.ShapeDtypeStruct((EP // 8, 8), jnp.float32),
    )(l2)
    w0, m = pl.pallas_call(
        _t_expmax_body,
        out_shape=[jax.ShapeDtypeStruct((EP // 8, 8), jnp.float32),
                   jax.ShapeDtypeStruct((1, 8), jnp.float32)],
    )(s)
    return w0.reshape(EP), s, m


def _t_mstab_body(d0_ref, d1_ref, m_ref, o_ref):
    den0 = d0_ref[...][0] + d1_ref[...][0]
    m = m_ref[0, 0]
    lse = jnp.log(den0[:, 0] + 1e-30) + m
    o_ref[...] = lse.reshape(NPAD // 128, 128)


def _t_mstab(den2, m):
    out = pl.pallas_call(
        _t_mstab_body,
        grid=(1,),
        in_specs=[
            pl.BlockSpec((1, NPAD, 16), lambda i: (0, 0, 0)),
            pl.BlockSpec((1, NPAD, 16), lambda i: (1, 0, 0)),
            pl.BlockSpec((1, 8), lambda i: (0, 0)),
        ],
        out_specs=pl.BlockSpec((NPAD // 128, 128), lambda i: (0, 0)),
        out_shape=jax.ShapeDtypeStruct((NPAD // 128, 128), jnp.float32),
    )(den2, den2, m)
    return out.reshape(NPAD)


def _t_exp2_body(lg_ref, ma_ref, w_ref):
    w_ref[...] = jnp.exp(lg_ref[...] - ma_ref[...])


def _t_exp2(lg, margs):
    ma2 = margs.reshape(EP // 8, 8)
    w = pl.pallas_call(
        _t_exp2_body,
        grid=(8,),
        in_specs=[pl.BlockSpec((EP // 64, 8), lambda i: (i, 0)),
                  pl.BlockSpec((EP // 64, 8), lambda i: (i, 0))],
        out_specs=pl.BlockSpec((EP // 64, 8), lambda i: (i, 0)),
        out_shape=jax.ShapeDtypeStruct((EP // 8, 8), jnp.float32),
    )(lg, ma2)
    return w.reshape(EP)


def _t_norm_proj_body(relu_out, ol_ref, oh_ref, b_ref, wl_ref, wr_ref,
                      ll_ref, lh_ref, rl_ref, rh_ref):
    lo = ol_ref[...]
    hi = oh_ref[...]
    den = hi[:, DEN_CH:DEN_CH + 1] + 1e-16
    o = (jnp.concatenate([lo, hi], axis=1) * _precise_recip(den)
         + b_ref[...])
    if relu_out:
        o = jnp.maximum(o, 0.0)
    mask = (lax.broadcasted_iota(jnp.int32, (1, CP), 1) < C)
    h = jnp.where(mask, o, 0.0)
    xl = _dot_bf16(h, wl_ref[...])
    xr = _dot_bf16(h, wr_ref[...])
    ll_ref[...] = xl[:, :128]
    lh_ref[...] = xl[:, 128:]
    rl_ref[...] = xr[:, :128]
    rh_ref[...] = xr[:, 128:]


def _t_norm_proj(relu_out, o_acc, bias_row, wl, wr):
    return pl.pallas_call(
        functools.partial(_t_norm_proj_body, relu_out),
        grid=(NPAD // 256,),
        in_specs=[
            pl.BlockSpec((256, 128), lambda i: (i, 0)),
            pl.BlockSpec((256, 128), lambda i: (i, 0)),
            pl.BlockSpec((1, CP), lambda i: (0, 0)),
            pl.BlockSpec((CP, CP), lambda i: (0, 0)),
            pl.BlockSpec((CP, CP), lambda i: (0, 0)),
        ],
        out_specs=[pl.BlockSpec((256, 128), lambda i: (i, 0))] * 4,
        out_shape=[jax.ShapeDtypeStruct((NPAD, 128), jnp.float32)] * 4,
    )(o_acc[0], o_acc[1], bias_row, wl, wr)


def _t_pair_proj_body(ol_ref, oh_ref, b_ref, w1a_ref, w1b_ref, b1_ref,
                      z0_ref, z1_ref):
    lo = ol_ref[...]
    hi = oh_ref[...]
    den = hi[:, DEN_CH:DEN_CH + 1] + 1e-16
    o = (jnp.concatenate([lo, hi], axis=1) * _precise_recip(den)
         + b_ref[...])
    mask = (lax.broadcasted_iota(jnp.int32, (1, CP), 1) < C)
    emb = jnp.where(mask, o, 0.0)
    z0_ref[...] = _dot_bf16(emb, w1a_ref[...]) + b1_ref[...]
    z1_ref[...] = _dot_bf16(emb, w1b_ref[...])


def _t_pair_proj(o_acc, bias_row, w1a, w1b, b1_row):
    return pl.pallas_call(
        _t_pair_proj_body,
        grid=(NPAD // 256,),
        in_specs=[
            pl.BlockSpec((256, 128), lambda i: (i, 0)),
            pl.BlockSpec((256, 128), lambda i: (i, 0)),
            pl.BlockSpec((1, CP), lambda i: (0, 0)),
            pl.BlockSpec((CP, H1), lambda i: (0, 0)),
            pl.BlockSpec((CP, H1), lambda i: (0, 0)),
            pl.BlockSpec((1, H1), lambda i: (0, 0)),
        ],
        out_specs=[pl.BlockSpec((256, H1), lambda i: (i, 0))] * 2,
        out_shape=[jax.ShapeDtypeStruct((NPAD, H1), jnp.float32)] * 2,
    )(o_acc[0], o_acc[1], bias_row, w1a, w1b, b1_row)


def _t_final_body(s_ref, b2_ref, o_ref):
    ss = _dot3(s_ref[...], _group_sum_mat())
    o_ref[...] = jnp.maximum(ss + b2_ref[...], 0.0)


def _t_final(s16, b2):
    s2 = s16.reshape(P // 8, 128)
    b2r = jnp.broadcast_to(b2.reshape(1, 1), (1, 8))
    out = pl.pallas_call(
        _t_final_body,
        grid=(4,),
        in_specs=[pl.BlockSpec((P // 32, 128), lambda i: (i, 0)),
                  pl.BlockSpec((1, 8), lambda i: (0, 0))],
        out_specs=pl.BlockSpec((P // 32, 8), lambda i: (i, 0)),
        out_shape=jax.ShapeDtypeStruct((P // 8, 8), jnp.float32),
    )(s2, b2r)
    return out.reshape(P, 1)


_MESH = plsc.VectorSubcoreMesh(core_axis_name="c", subcore_axis_name="s")


@functools.partial(
    pl.kernel,
    mesh=_MESH,
    out_type=jax.ShapeDtypeStruct((EP, 16), jnp.float32),
    scratch_types=[
        pltpu.VMEM((TE1,), jnp.int32),
        pltpu.VMEM((TE1,), jnp.int32),
        pltpu.VMEM((G1, 16), jnp.float32),
        pltpu.VMEM((G1, 16), jnp.float32),
        pltpu.VMEM((G1, 128), jnp.float32),
        pltpu.VMEM((G1, 128), jnp.float32),
        pltpu.VMEM((G1, 128), jnp.float32),
        pltpu.VMEM((G1, 128), jnp.float32),
        pltpu.VMEM((G1, 128), jnp.float32),
        pltpu.VMEM((G1, 128), jnp.float32),
        pltpu.VMEM((G1, 128), jnp.float32),
        pltpu.VMEM((G1, 128), jnp.float32),
        pltpu.VMEM((CP,), jnp.float32),
        pltpu.SemaphoreType.DMA,
        pltpu.SemaphoreType.DMA,
        pltpu.SemaphoreType.DMA,
        pltpu.SemaphoreType.DMA,
    ],
)
def _sc_logits(xl_lo, xl_hi, xr_lo, xr_hi, src_hbm, dst_hbm, att_hbm,
               out_hbm, src_v, dst_v, lg0, lg1,
               bll0, blh0, brl0, brh0, bll1, blh1, brl1, brh1, att_v,
               semg0, semg1, semw0, semw1):
    c = lax.axis_index("c")
    s = lax.axis_index("s")
    wid = s * NC + c
    base = wid * TE1
    pltpu.sync_copy(src_hbm.at[pl.ds(base, TE1)], src_v)
    pltpu.sync_copy(dst_hbm.at[pl.ds(base, TE1)], dst_v)
    pltpu.sync_copy(att_hbm, att_v)

    bufsets = ((bll0, blh0, brl0, brh0, semg0, lg0, semw0),
               (bll1, blh1, brl1, brh1, semg1, lg1, semw1))
    tabs = (xl_lo, xl_hi, xr_lo, xr_hi)

    def issue(ci, bs):
        off = ci * G1
        for t in range(4):
            idx = src_v if t < 2 else dst_v
            pltpu.async_copy(tabs[t].at[idx.at[pl.ds(off, G1)]], bs[t], bs[4])

    def drain(bs):
        for t in range(4):
            idx = src_v if t < 2 else dst_v
            pltpu.make_async_copy(tabs[t].at[idx.at[pl.ds(0, G1)]], bs[t],
                                  bs[4]).wait()

    issue(0, bufsets[0])

    def outer(i, carry):
        ci0 = i * 2
        for b in range(2):
            bs = bufsets[b]
            ci = ci0 + b
            nci = ci + 1

            @pl.when(nci < CH1)
            def _():
                issue(nci, bufsets[1 - b])

            drain(bs)
            lg_c = bs[5]

            @pl.when(ci >= 2)
            def _():
                pltpu.make_async_copy(
                    lg_c, out_hbm.at[pl.ds(base, G1)], bs[6]).wait()

            def edge(e, carry2):
                acc = jnp.zeros((16,), jnp.float32)
                for k in range(8):
                    sl = pl.ds(16 * k, 16)
                    sh = pl.ds(128 + 16 * k, 16)
                    v = bs[0][e, sl] + bs[2][e, sl]
                    acc = acc + att_v[sl] * _round_bf16(_leaky(v))
                    v2 = bs[1][e, sl] + bs[3][e, sl]
                    acc = acc + att_v[sh] * _round_bf16(_leaky(v2))
                lg_c[e, :] = acc
                return carry2

            lax.fori_loop(0, G1, edge, 0)
            pltpu.async_copy(lg_c, out_hbm.at[pl.ds(base + ci * G1, G1)],
                             bs[6])
        return carry

    lax.fori_loop(0, CH1 // 2, outer, 0)
    for b in range(2):
        bs = bufsets[b]
        pltpu.make_async_copy(bs[5], out_hbm.at[pl.ds(base, G1)],
                              bs[6]).wait()


@functools.partial(
    pl.kernel,
    mesh=_MESH,
    out_type=jax.ShapeDtypeStruct((NC, NPAD, 128), jnp.float32),
    scratch_types=[
        pltpu.VMEM((G2,), jnp.int32),
        pltpu.VMEM((G2,), jnp.int32),
        pltpu.VMEM((G2,), jnp.int32),
        pltpu.VMEM((G2,), jnp.int32),
        pltpu.VMEM((G2,), jnp.float32),
        pltpu.VMEM((G2,), jnp.float32),
        pltpu.VMEM((G2, 128), jnp.float32),
        pltpu.VMEM((G2, 128), jnp.float32),
        pltpu.VMEM_SHARED((NPAD, 128), jnp.float32),
        pltpu.SemaphoreType.DMA,
        pltpu.SemaphoreType.DMA,
        pltpu.SemaphoreType.DMA,
        pltpu.SemaphoreType.DMA,
        pltpu.SemaphoreType.DMA,
        pltpu.SemaphoreType.DMA,
        pltpu.SemaphoreType.DMA,
        pltpu.SemaphoreType.DMA,
    ],
)
def _sc_scatter(xl_lo, xl_hi, src_hbm, dst3d_hbm, w_hbm, out_hbm,
                sb0, sb1, db0, db1, wb0, wb1, rows0, rows1, shared,
                sem_g0, sem_g1, sem_s0, sem_s1, sem_d0, sem_d1,
                sem_w0, sem_w1):
    c = lax.axis_index("c")
    s = lax.axis_index("s")
    ebase = s * TE2

    def zrow(e, carry):
        for k in range(8):
            rows0[e, pl.ds(16 * k, 16)] = jnp.zeros((16,), jnp.float32)
        return carry

    lax.fori_loop(0, G2, zrow, 0)
    nbase = s * ROWS_PER_TILE
    for i in range(ROWS_PER_TILE // G2):
        pltpu.sync_copy(rows0, shared.at[pl.ds(nbase + i * G2, G2)])
    plsc.subcore_barrier()

    lane = lax.broadcasted_iota(jnp.int32, (16,), 0)
    sbufs = (sb0, sb1)
    dbufs = (db0, db1)
    wbufs = (wb0, wb1)
    rbufs = (rows0, rows1)
    gsems = (sem_g0, sem_g1)
    ssems = (sem_s0, sem_s1)
    dsems = (sem_d0, sem_d1)
    wsems = (sem_w0, sem_w1)

    def run(tab, is_hi):
        def issue_src(ci, b):
            pltpu.async_copy(src_hbm.at[pl.ds(ebase + ci * G2, G2)],
                             sbufs[b], ssems[b])

        def issue_aux(ci, b):
            pltpu.async_copy(w_hbm.at[pl.ds(ebase + ci * G2, G2)],
                             wbufs[b], wsems[b])
            pltpu.async_copy(dst3d_hbm.at[s].at[ci], dbufs[b], dsems[b])

        def issue_gather(b, rb):
            pltpu.async_copy(tab.at[sbufs[b]], rbufs[rb], gsems[rb])

        pltpu.sync_copy(src_hbm.at[pl.ds(ebase, G2)], sb0)
        issue_gather(0, 0)
        issue_src(1, 1)
        issue_aux(0, 0)

        def outer(i, carry):
            ci0 = i * 2
            for b in range(2):
                ci = ci0 + b
                nci = ci + 1
                rows = rbufs[b]

                pltpu.make_async_copy(tab.at[sbufs[b]], rows,
                                      gsems[b]).wait()

                @pl.when(ci + 2 < CH2)
                def _():
                    issue_src(ci + 2, b)

                @pl.when(nci < CH2)
                def _():
                    pltpu.make_async_copy(
                        src_hbm.at[pl.ds(ebase, G2)], sbufs[1 - b],
                        ssems[1 - b]).wait()
                    issue_gather(1 - b, 1 - b)
                    issue_aux(nci, 1 - b)

                pltpu.make_async_copy(w_hbm.at[pl.ds(ebase, G2)],
                                      wbufs[b], wsems[b]).wait()

                def grp(g, carry2):
                    wrow = wbufs[b][pl.ds(g * 16, 16)]
                    for j in range(16):
                        e = g * 16 + j
                        wv = wrow[j]
                        for k in range(8):
                            sl = pl.ds(16 * k, 16)
                            scaled = rows[e, sl] * wv
                            if is_hi and k == 7:
                                scaled = jnp.where(lane == (DEN_CH - 112),
                                                   wv, scaled)
                            rows[e, sl] = scaled
                    return carry2

                lax.fori_loop(0, G2 // 16, grp, 0)

                pltpu.make_async_copy(dst3d_hbm.at[s].at[0], dbufs[b],
                                      dsems[b]).wait()
                pltpu.sync_copy(rows, shared.at[dbufs[b]], add=True)
            return carry

        lax.fori_loop(0, CH2 // 2, outer, 0)

    @pl.when(c == 0)
    def _():
        run(xl_lo, False)

    @pl.when(c == 1)
    def _():
        run(xl_hi, True)

    plsc.subcore_barrier()
    pltpu.sync_copy(shared.at[pl.ds(nbase, ROWS_PER_TILE)],
                    out_hbm.at[c].at[pl.ds(nbase, ROWS_PER_TILE)])


@functools.partial(
    pl.kernel,
    mesh=_MESH,
    out_type=jax.ShapeDtypeStruct((P, 16), jnp.float32),
    scratch_types=[
        pltpu.VMEM((TP,), jnp.int32),
        pltpu.VMEM((TP,), jnp.int32),
        pltpu.VMEM((G3, 16), jnp.float32),
        pltpu.VMEM((G3, 16), jnp.float32),
        pltpu.VMEM((G3, CP), jnp.float32),
        pltpu.VMEM((G3, CP), jnp.float32),
        pltpu.VMEM((G3, CP), jnp.float32),
        pltpu.VMEM((G3, CP), jnp.float32),
        pltpu.VMEM((CP,), jnp.float32),
        pltpu.SemaphoreType.DMA,
        pltpu.SemaphoreType.DMA,
        pltpu.SemaphoreType.DMA,
        pltpu.SemaphoreType.DMA,
    ],
)
def _sc_pairs(z0_hbm, z1_hbm, p0_hbm, p1_hbm, w2_hbm, out_hbm,
              p0_v, p1_v, o0, o1, r00, r10, r01, r11, w2_v,
              semg0, semg1, semw0, semw1):
    c = lax.axis_index("c")
    s = lax.axis_index("s")
    wid = s * NC + c
    base = wid * TP
    pltpu.sync_copy(p0_hbm.at[pl.ds(base, TP)], p0_v)
    pltpu.sync_copy(p1_hbm.at[pl.ds(base, TP)], p1_v)
    pltpu.sync_copy(w2_hbm, w2_v)

    bufsets = ((r00, r10, semg0, o0, semw0), (r01, r11, semg1, o1, semw1))

    def issue(ci, bs):
        off = ci * G3
        pltpu.async_copy(z0_hbm.at[p0_v.at[pl.ds(off, G3)]], bs[0], bs[2])
        pltpu.async_copy(z1_hbm.at[p1_v.at[pl.ds(off, G3)]], bs[1], bs[2])

    def drain(bs):
        pltpu.make_async_copy(z0_hbm.at[p0_v.at[pl.ds(0, G3)]], bs[0],
                              bs[2]).wait()
        pltpu.make_async_copy(z1_hbm.at[p1_v.at[pl.ds(0, G3)]], bs[1],
                              bs[2]).wait()

    issue(0, bufsets[0])

    def outer(i, carry):
        ci0 = i * 2
        for b in range(2):
            bs = bufsets[b]
            ci = ci0 + b
            nci = ci + 1

            @pl.when(nci < CH3)
            def _():
                issue(nci, bufsets[1 - b])

            drain(bs)
            o_c = bs[3]

            @pl.when(ci >= 2)
            def _():
                pltpu.make_async_copy(
                    o_c, out_hbm.at[pl.ds(base, G3)], bs[4]).wait()

            def pair(e, carry2):
                acc = jnp.zeros((16,), jnp.float32)
                for k in range(16):
                    sl = pl.ds(16 * k, 16)
                    v = _round_bf16(
                        jnp.maximum(bs[0][e, sl] + bs[1][e, sl], 0.0))
                    acc = acc + w2_v[sl] * v
                o_c[e, :] = acc
                return carry2

            lax.fori_loop(0, G3, pair, 0)
            pltpu.async_copy(o_c, out_hbm.at[pl.ds(base + ci * G3, G3)],
                             bs[4])
        return carry

    lax.fori_loop(0, CH3 // 2, outer, 0)
    for b in range(2):
        bs = bufsets[b]
        pltpu.make_async_copy(bs[3], out_hbm.at[pl.ds(base, G3)],
                              bs[4]).wait()


@functools.partial(
    pl.kernel,
    mesh=_MESH,
    out_type=jax.ShapeDtypeStruct((NC, NPAD, 16), jnp.float32),
    scratch_types=[
        pltpu.VMEM((TE1 // G1, G1), jnp.int32),
        pltpu.VMEM((TE1,), jnp.float32),
        pltpu.VMEM((G1, 16), jnp.float32),
        pltpu.VMEM_SHARED((NPAD, 16), jnp.float32),
    ],
)
def _sc_denscat(dst3dA_hbm, w_hbm, out_hbm, dst_v, w_v, rows, shared):
    c = lax.axis_index("c")
    s = lax.axis_index("s")
    wid = s * NC + c
    base = wid * TE1
    pltpu.sync_copy(dst3dA_hbm.at[wid], dst_v)
    pltpu.sync_copy(w_hbm.at[pl.ds(base, TE1)], w_v)

    lane = lax.broadcasted_iota(jnp.int32, (16,), 0)
    nrows = NPAD // NS

    def zrow(e, carry):
        rows[e, :] = jnp.zeros((16,), jnp.float32)
        return carry

    lax.fori_loop(0, G1, zrow, 0)
    nbase = s * nrows
    for i in range(nrows // G1):
        pltpu.sync_copy(rows, shared.at[pl.ds(nbase + i * G1, G1)])
    plsc.subcore_barrier()

    def chunk(ci, carry):
        off = ci * G1

        def grp(g, carry2):
            wrow = w_v[pl.ds(off + g * 16, 16)]
            for j in range(16):
                rows[g * 16 + j, :] = jnp.where(lane == 0, wrow[j], 0.0)
            return carry2

        lax.fori_loop(0, G1 // 16, grp, 0)
        pltpu.sync_copy(rows, shared.at[dst_v.at[ci]], add=True)
        return carry

    lax.fori_loop(0, TE1 // G1, chunk, 0)
    plsc.subcore_barrier()
    pltpu.sync_copy(shared.at[pl.ds(nbase, nrows)],
                    out_hbm.at[c].at[pl.ds(nbase, nrows)])


@functools.partial(
    pl.kernel,
    mesh=_MESH,
    out_type=jax.ShapeDtypeStruct((EP,), jnp.float32),
    scratch_types=[
        pltpu.VMEM((TE1,), jnp.int32),
        pltpu.VMEM((G1,), jnp.float32),
        pltpu.VMEM((G1,), jnp.float32),
        pltpu.SemaphoreType.DMA,
        pltpu.SemaphoreType.DMA,
    ],
)
def _sc_mgather(m1_hbm, dst_hbm, out_hbm, dst_v, b0, b1, sem0, sem1):
    c = lax.axis_index("c")
    s = lax.axis_index("s")
    wid = s * NC + c
    base = wid * TE1
    pltpu.sync_copy(dst_hbm.at[pl.ds(base, TE1)], dst_v)
    bufs = (b0, b1)
    sems = (sem0, sem1)

    def issue(ci, b):
        pltpu.async_copy(m1_hbm.at[dst_v.at[pl.ds(ci * G1, G1)]],
                         bufs[b], sems[b])

    issue(0, 0)

    def outer(i, carry):
        ci0 = i * 2
        for b in range(2):
            ci = ci0 + b

            @pl.when(ci + 1 < TE1 // G1)
            def _():
                issue(ci + 1, 1 - b)

            pltpu.make_async_copy(m1_hbm.at[dst_v.at[pl.ds(0, G1)]],
                                  bufs[b], sems[b]).wait()
            pltpu.sync_copy(bufs[b], out_hbm.at[pl.ds(base + ci * G1, G1)])
        return carry

    lax.fori_loop(0, TE1 // G1 // 2, outer, 0)


def kernel(x, edge_index, pairs, Wl1, Wr1, att1, bc1, Wl2, Wr2, att2, bc2,
           W1, b1, W2, b2):
    loop = jnp.arange(N, dtype=jnp.int32)
    padi = jnp.full((EP - E - N,), DUMMY, jnp.int32)
    src = jnp.concatenate([edge_index[0], loop, padi])
    dst = jnp.concatenate([edge_index[1], loop, padi])
    dst3d = dst.reshape(NS, CH2, G2)
    x_pad = jnp.pad(x, ((0, NPAD - N), (0, 0)))

    def padw(w):
        return jnp.pad(w, ((0, 0), (0, CP - C)))

    wl1 = padw(Wl1)
    wr1 = padw(Wr1)
    def bf16r(a):
        return a.astype(jnp.bfloat16).astype(jnp.float32)

    att1p = bf16r(jnp.pad(att1, (0, CP - C)))
    att2p = bf16r(jnp.pad(att2, (0, CP - C)))
    bc1r = jnp.pad(bc1, (0, CP - C)).reshape(1, CP)
    bc2r = jnp.pad(bc2, (0, CP - C)).reshape(1, CP)
    wl2 = jnp.pad(Wl2, ((0, CP - C), (0, CP - C)))
    wr2 = jnp.pad(Wr2, ((0, CP - C), (0, CP - C)))
    w1a = jnp.pad(W1[:C], ((0, CP - C), (0, 0)))
    w1b = jnp.pad(W1[C:], ((0, CP - C), (0, 0)))
    b1r = b1.reshape(1, H1)
    w2v = bf16r(W2[:, 0])
    p0 = pairs[:, 0]
    p1 = pairs[:, 1]

    dst3dA = dst.reshape(NW, TE1 // G1, G1)

    def softmax_w(logits16):
        w0, lg, m = _t_exp(logits16)
        den2 = _sc_denscat(dst3dA, w0)
        m1 = _t_mstab(den2, m)
        margs = _sc_mgather(m1, dst)
        return _t_exp2(lg, margs)

    ll1, lh1, rl1, rh1 = _t_proj(x_pad, wl1, wr1)
    logits1 = _sc_logits(ll1, lh1, rl1, rh1, src, dst, att1p)
    w1e = softmax_w(logits1)
    acc1 = _sc_scatter(ll1, lh1, src, dst3d, w1e)

    ll2, lh2, rl2, rh2 = _t_norm_proj(True, acc1, bc1r, wl2, wr2)
    logits2 = _sc_logits(ll2, lh2, rl2, rh2, src, dst, att2p)
    w2e = softmax_w(logits2)
    acc2 = _sc_scatter(ll2, lh2, src, dst3d, w2e)

    z0, z1 = _t_pair_proj(acc2, bc2r, w1a, w1b, b1r)
    s16 = _sc_pairs(z0, z1, p0, p1, w2v)
    return _t_final(s16, b2)
